# baseline (device time: 58890 ns/iter reference)
import jax
import jax.numpy as jnp
from jax import lax
from jax.experimental import pallas as pl
from jax.experimental.pallas import tpu as pltpu

N_DEV = 4
B = 2
SQ = 512
SKV = 512
HQ = 32
HG = HQ // N_DEV
DH = 64
DM = 768
WCH = HG * DH
BLK = 64
F32 = jnp.float32
BF16 = jnp.bfloat16


def kernel(x, Wq, K_ext, V_ext, Wo):
    x2d = x.reshape(B * SQ, DM)
    k2 = K_ext.reshape(B, SKV, HQ * DH)
    v2 = V_ext.reshape(B, SKV, HQ * DH)

    def body(x_ref, wq_ref, k_ref, v_ref, wo_ref, out_ref,
             wq_slot, wo_slot, k_slot, v_slot, send_sems, recv_sems):
        my_pos = lax.axis_index("i")
        left = (my_pos + N_DEV - 1) % N_DEV
        right = (my_pos + 1) % N_DEV

        barrier_sem = pltpu.get_barrier_semaphore()
        for nbr in (left, right):
            pl.semaphore_signal(
                barrier_sem, inc=1,
                device_id=(nbr,), device_id_type=pl.DeviceIdType.MESH,
            )
        pl.semaphore_wait(barrier_sem, 2)

        wq_slot[:, 0:WCH] = wq_ref[...].astype(BF16)
        wo_slot[0:WCH, :] = wo_ref[...].astype(BF16)

        def rc(src, dst, i, dev):
            return pltpu.make_async_remote_copy(
                src_ref=src, dst_ref=dst,
                send_sem=send_sems.at[i], recv_sem=recv_sems.at[i],
                device_id=(dev,), device_id_type=pl.DeviceIdType.MESH,
            )

        h1qr = rc(wq_slot.at[:, 0:WCH], wq_slot.at[:, 3 * WCH:4 * WCH], 0, right)
        h1or = rc(wo_slot.at[0:WCH, :], wo_slot.at[3 * WCH:4 * WCH, :], 1, right)
        h1ql = rc(wq_slot.at[:, 0:WCH], wq_slot.at[:, 1 * WCH:2 * WCH], 2, left)
        h1ol = rc(wo_slot.at[0:WCH, :], wo_slot.at[1 * WCH:2 * WCH, :], 3, left)
        h2q = rc(wq_slot.at[:, 3 * WCH:4 * WCH], wq_slot.at[:, 2 * WCH:3 * WCH], 4, right)
        h2o = rc(wo_slot.at[1 * WCH:2 * WCH, :], wo_slot.at[2 * WCH:3 * WCH, :], 5, left)

        h1qr.start()
        h1ql.start()
        h1or.start()
        h1ol.start()

        for j in range(N_DEV):
            s_j = (j - my_pos) % N_DEV
            k_slot[:, :, pl.ds(s_j * WCH, WCH)] = \
                k_ref[:, :, j * WCH:(j + 1) * WCH].astype(BF16)
            v_slot[:, :, pl.ds(s_j * WCH, WCH)] = \
                v_ref[:, :, j * WCH:(j + 1) * WCH].astype(BF16)

        xb = (x_ref[...] * 0.125).astype(BF16)
        qb = my_pos * (SQ // BLK) + \
            lax.broadcasted_iota(jnp.int32, (SQ, SKV), 0) // BLK
        kb = lax.broadcasted_iota(jnp.int32, (SQ, SKV), 1) // BLK
        mask = (qb == kb) | (kb == 0) | ((qb + kb) % 3 == 0)
        neg = jnp.where(mask, 0.0, -1e9).astype(F32)

        def attn(s):
            qg = jnp.dot(xb, wq_slot[:, s * WCH:(s + 1) * WCH],
                         preferred_element_type=F32).astype(BF16)
            ctx_b = []
            for b in range(B):
                ctxs = []
                for hh in range(HG):
                    c0 = s * WCH + hh * DH
                    q = qg[b * SQ:(b + 1) * SQ, hh * DH:(hh + 1) * DH]
                    k = k_slot[b, :, c0:c0 + DH]
                    sc = lax.dot_general(
                        q, k, (((1,), (1,)), ((), ())),
                        preferred_element_type=F32,
                    )
                    w = jnp.exp((sc + neg).astype(BF16))
                    denom = jnp.sum(w, axis=1, keepdims=True, dtype=F32)
                    cu = jnp.dot(w, v_slot[b, :, c0:c0 + DH],
                                 preferred_element_type=F32)
                    ctxs.append(cu * (1.0 / denom))
                ctx_b.append(jnp.concatenate(ctxs, axis=1).astype(BF16))
            return ctx_b

        def contrib(s, ctx_b, first):
            wo_s = wo_slot[s * WCH:(s + 1) * WCH, :]
            for b in range(B):
                c = jnp.dot(ctx_b[b], wo_s, preferred_element_type=F32)
                if first:
                    out_ref[b] = c
                else:
                    out_ref[b] = out_ref[b] + c

        contrib(0, attn(0), True)

        h1qr.wait_recv()
        h1ql.wait_recv()
        h2q.start()
        ctx3 = attn(3)
        h1or.wait_recv()
        contrib(3, ctx3, False)
        ctx1 = attn(1)
        h1ol.wait_recv()
        h2o.start()
        contrib(1, ctx1, False)

        h2q.wait_recv()
        ctx2 = attn(2)
        h2o.wait_recv()
        contrib(2, ctx2, False)

        for d in (h1qr, h1or, h1ql, h1ol, h2q, h2o):
            d.wait_send()

    return pl.pallas_call(
        body,
        out_shape=jax.ShapeDtypeStruct((B, SQ, DM), F32),
        in_specs=[pl.BlockSpec(memory_space=pltpu.VMEM)] * 5,
        out_specs=pl.BlockSpec(memory_space=pltpu.VMEM),
        scratch_shapes=[
            pltpu.VMEM((DM, HQ * DH), BF16),
            pltpu.VMEM((HQ * DH, DM), BF16),
            pltpu.VMEM((B, SKV, HQ * DH), BF16),
            pltpu.VMEM((B, SKV, HQ * DH), BF16),
            pltpu.SemaphoreType.DMA((6,)),
            pltpu.SemaphoreType.DMA((6,)),
        ],
        compiler_params=pltpu.CompilerParams(collective_id=0),
    )(x2d, Wq, k2, v2, Wo)


# device time: 58717 ns/iter; 1.0029x vs baseline; 1.0029x over previous
import jax
import jax.numpy as jnp
from jax import lax
from jax.experimental import pallas as pl
from jax.experimental.pallas import tpu as pltpu

N_DEV = 4
B = 2
SQ = 512
SKV = 512
HQ = 32
HG = HQ // N_DEV
DH = 64
DM = 768
WCH = HG * DH
BLK = 64
F32 = jnp.float32
BF16 = jnp.bfloat16


def kernel(x, Wq, K_ext, V_ext, Wo):
    x2d = x.reshape(B * SQ, DM)
    k2 = K_ext.reshape(B, SKV, HQ * DH)
    v2 = V_ext.reshape(B, SKV, HQ * DH)

    def body(x_ref, wq_ref, k_ref, v_ref, wo_ref, out_ref,
             wq_slot, wo_slot, k_slot, v_slot, send_sems, recv_sems):
        my_pos = lax.axis_index("i")
        left = (my_pos + N_DEV - 1) % N_DEV
        right = (my_pos + 1) % N_DEV

        barrier_sem = pltpu.get_barrier_semaphore()
        for nbr in (left, right):
            pl.semaphore_signal(
                barrier_sem, inc=1,
                device_id=(nbr,), device_id_type=pl.DeviceIdType.MESH,
            )
        pl.semaphore_wait(barrier_sem, 2)

        wq_slot[:, 0:WCH] = wq_ref[...].astype(BF16)
        wo_slot[0:WCH, :] = wo_ref[...].astype(BF16)

        def rc(src, dst, i, dev):
            return pltpu.make_async_remote_copy(
                src_ref=src, dst_ref=dst,
                send_sem=send_sems.at[i], recv_sem=recv_sems.at[i],
                device_id=(dev,), device_id_type=pl.DeviceIdType.MESH,
            )

        h1qr = rc(wq_slot.at[:, 0:WCH], wq_slot.at[:, 3 * WCH:4 * WCH], 0, right)
        h1or = rc(wo_slot.at[0:WCH, :], wo_slot.at[3 * WCH:4 * WCH, :], 1, right)
        h1ql = rc(wq_slot.at[:, 0:WCH], wq_slot.at[:, 1 * WCH:2 * WCH], 2, left)
        h1ol = rc(wo_slot.at[0:WCH, :], wo_slot.at[1 * WCH:2 * WCH, :], 3, left)
        h2q = rc(wq_slot.at[:, 3 * WCH:4 * WCH], wq_slot.at[:, 2 * WCH:3 * WCH], 4, right)
        h2o = rc(wo_slot.at[1 * WCH:2 * WCH, :], wo_slot.at[2 * WCH:3 * WCH, :], 5, left)

        h1qr.start()
        h1ql.start()
        h1or.start()
        h1ol.start()

        for j in range(N_DEV):
            s_j = (j - my_pos) % N_DEV
            k_slot[:, :, pl.ds(s_j * WCH, WCH)] = \
                k_ref[:, :, j * WCH:(j + 1) * WCH].astype(BF16)
            v_slot[:, :, pl.ds(s_j * WCH, WCH)] = \
                v_ref[:, :, j * WCH:(j + 1) * WCH].astype(BF16)

        xb = (x_ref[...] * 0.125).astype(BF16)
        qb = my_pos * (SQ // BLK) + \
            lax.broadcasted_iota(jnp.int32, (SQ, SKV), 0) // BLK
        kb = lax.broadcasted_iota(jnp.int32, (SQ, SKV), 1) // BLK
        mask = (qb == kb) | (kb == 0) | ((qb + kb) % 3 == 0)
        neg = jnp.where(mask, 0.0, -1e9).astype(F32)

        def attn(s):
            qg = jnp.dot(xb, wq_slot[:, s * WCH:(s + 1) * WCH],
                         preferred_element_type=F32).astype(BF16)
            ctx_b = []
            for b in range(B):
                ctxs = []
                for hh in range(HG):
                    c0 = s * WCH + hh * DH
                    q = qg[b * SQ:(b + 1) * SQ, hh * DH:(hh + 1) * DH]
                    k = k_slot[b, :, c0:c0 + DH]
                    sc = lax.dot_general(
                        q, k, (((1,), (1,)), ((), ())),
                        preferred_element_type=F32,
                    )
                    w = jnp.exp(sc + neg)
                    denom = jnp.sum(w, axis=1, keepdims=True)
                    cu = jnp.dot(w.astype(BF16), v_slot[b, :, c0:c0 + DH],
                                 preferred_element_type=F32)
                    ctxs.append(cu * (1.0 / denom))
                ctx_b.append(jnp.concatenate(ctxs, axis=1).astype(BF16))
            return ctx_b

        def contrib(s, ctx_b, first):
            wo_s = wo_slot[s * WCH:(s + 1) * WCH, :]
            for b in range(B):
                c = jnp.dot(ctx_b[b], wo_s, preferred_element_type=F32)
                if first:
                    out_ref[b] = c
                else:
                    out_ref[b] = out_ref[b] + c

        contrib(0, attn(0), True)

        h1qr.wait_recv()
        h1ql.wait_recv()
        h2q.start()
        ctx3 = attn(3)
        h1or.wait_recv()
        contrib(3, ctx3, False)
        ctx1 = attn(1)
        h1ol.wait_recv()
        h2o.start()
        contrib(1, ctx1, False)

        h2q.wait_recv()
        ctx2 = attn(2)
        h2o.wait_recv()
        contrib(2, ctx2, False)

        for d in (h1qr, h1or, h1ql, h1ol, h2q, h2o):
            d.wait_send()

    return pl.pallas_call(
        body,
        out_shape=jax.ShapeDtypeStruct((B, SQ, DM), F32),
        in_specs=[pl.BlockSpec(memory_space=pltpu.VMEM)] * 5,
        out_specs=pl.BlockSpec(memory_space=pltpu.VMEM),
        scratch_shapes=[
            pltpu.VMEM((DM, HQ * DH), BF16),
            pltpu.VMEM((HQ * DH, DM), BF16),
            pltpu.VMEM((B, SKV, HQ * DH), BF16),
            pltpu.VMEM((B, SKV, HQ * DH), BF16),
            pltpu.SemaphoreType.DMA((6,)),
            pltpu.SemaphoreType.DMA((6,)),
        ],
        compiler_params=pltpu.CompilerParams(collective_id=0),
    )(x2d, Wq, k2, v2, Wo)
